# baseline (speedup 1.0000x reference)
import functools

import jax
import jax.numpy as jnp
from jax import lax
from jax.experimental import pallas as pl
from jax.experimental.pallas import tpu as pltpu
from jax.experimental.pallas import tpu_sc as plsc

NC = 2
NS = 16
NW = NC * NS
F32 = jnp.float32


def _make_deg_kernel(np_, epw):
    n16 = epw // 16
    mesh = plsc.VectorSubcoreMesh(core_axis_name="c", subcore_axis_name="s")

    @functools.partial(
        pl.kernel,
        out_type=jax.ShapeDtypeStruct((NW, np_), F32),
        mesh=mesh,
        scratch_types=[
            pltpu.VMEM((epw,), jnp.int32),
            pltpu.VMEM((np_,), F32),
        ],
        compiler_params=pltpu.CompilerParams(needs_layout_passes=False),
    )
    def deg_kernel(dst2, out_hbm, dstv, hist):
        c = lax.axis_index("c")
        s = lax.axis_index("s")
        wid = c * NS + s
        pltpu.sync_copy(dst2.at[wid], dstv)
        zero16 = jnp.zeros((16,), F32)
        ones16 = jnp.ones((16,), F32)

        def zbody(i, carry):
            hist[pl.ds(i * 16, 16)] = zero16
            return carry

        lax.fori_loop(0, np_ // 16, zbody, 0)

        def body(i, carry):
            idx16 = dstv[pl.ds(i * 16, 16)]
            plsc.addupdate_scatter(hist, [idx16], ones16)
            return carry

        lax.fori_loop(0, n16, body, 0)
        pltpu.sync_copy(hist, out_hbm.at[wid])

    return deg_kernel


def _make_edge_agg_kernel(np_, n_chunks, feat, grp=8):
    rpt = np_ // NS
    chunk = 128
    n_grp = n_chunks // grp
    co = 128
    n_co = rpt // co
    mesh = plsc.VectorSubcoreMesh(core_axis_name="c", subcore_axis_name="s")

    @functools.partial(
        pl.kernel,
        out_type=jax.ShapeDtypeStruct((NC, np_, feat), F32),
        mesh=mesh,
        scratch_types=[
            pltpu.VMEM((n_chunks, chunk), jnp.int32),
            pltpu.VMEM((grp, chunk), jnp.int32),
            pltpu.VMEM((2, chunk, feat), F32),
            pltpu.VMEM_SHARED((np_, feat), F32),
            pltpu.SemaphoreType.DMA,
            pltpu.SemaphoreType.DMA,
            pltpu.SemaphoreType.DMA,
            pltpu.SemaphoreType.DMA,
        ],
    )
    def agg_kernel(g_hbm, src3, dst3, zeros_hbm, out_hbm,
                   idx_s, idx_d, rows, acc, sg0, sg1, ss0, ss1):
        c = lax.axis_index("c")
        s = lax.axis_index("s")
        r0 = pl.multiple_of(s * rpt, 128)
        pltpu.sync_copy(zeros_hbm.at[pl.ds(r0, rpt)], acc.at[pl.ds(r0, rpt)])
        wid = c * NS + s
        pltpu.sync_copy(src3.at[wid], idx_s)
        sg = (sg0, sg1)
        ss = (ss0, ss1)

        gd = [None, None]
        gd[0] = pltpu.async_copy(g_hbm.at[idx_s.at[0]], rows.at[0], sg0)
        plsc.subcore_barrier()

        def group(jo, carry):
            j0 = pl.multiple_of(jo * grp, grp)
            pltpu.sync_copy(dst3.at[wid, pl.ds(j0, grp)], idx_d)
            sd = [None] * grp
            for k in range(grp):
                b = k % 2
                if k >= 1:
                    sd[k - 1].wait()
                if k < grp - 1:
                    gd[1 - b] = pltpu.async_copy(
                        g_hbm.at[idx_s.at[jo * grp + k + 1]],
                        rows.at[1 - b], sg[1 - b])
                else:
                    @pl.when(jo < n_grp - 1)
                    def _pref():
                        pltpu.async_copy(
                            g_hbm.at[idx_s.at[jo * grp + grp]],
                            rows.at[1 - b], sg[1 - b])
                gd[b].wait()
                sd[k] = pltpu.async_copy(
                    rows.at[b], acc.at[idx_d.at[k]], ss[b], add=True)
            sd[grp - 1].wait()
            return carry

        lax.fori_loop(0, n_grp, group, 0)
        plsc.subcore_barrier()
        pltpu.sync_copy(acc.at[pl.ds(r0, rpt)], out_hbm.at[c, pl.ds(r0, rpt)])

    return agg_kernel


def _matmul(xp, W1, bm):
    n, f = xp.shape
    grid = n // bm
    hid = W1.shape[1]

    def body(x_ref, w_ref, h_ref):
        h_ref[...] = jnp.dot(x_ref[...], w_ref[...],
                             preferred_element_type=F32)

    return pl.pallas_call(
        body,
        grid=(grid,),
        in_specs=[
            pl.BlockSpec((bm, f), lambda i: (i, 0)),
            pl.BlockSpec((f, hid), lambda i: (0, 0)),
        ],
        out_specs=pl.BlockSpec((bm, hid), lambda i: (i, 0)),
        out_shape=jax.ShapeDtypeStruct((n, hid), F32),
    )(xp, W1)


def _stage1(deg_p, h, bm):
    n, hid = h.shape
    grid = n // bm

    def body(degp_ref, h_ref, g1_ref, dinv_ref):
        deg = lax.dot_general(
            degp_ref[...], jnp.ones((NW, 1), F32), (((0,), (0,)), ((), ())),
            preferred_element_type=F32) + 1.0
        dinv = lax.rsqrt(deg)
        g1_ref[...] = h_ref[...] * dinv
        dinv_ref[...] = dinv

    return pl.pallas_call(
        body,
        grid=(grid,),
        in_specs=[
            pl.BlockSpec((NW, bm), lambda i: (0, i)),
            pl.BlockSpec((bm, hid), lambda i: (i, 0)),
        ],
        out_specs=[
            pl.BlockSpec((bm, hid), lambda i: (i, 0)),
            pl.BlockSpec((bm, 1), lambda i: (i, 0)),
        ],
        out_shape=[
            jax.ShapeDtypeStruct((n, hid), F32),
            jax.ShapeDtypeStruct((n, 1), F32),
        ],
    )(deg_p, h)


def _stage2(s1_p, g1, dinv, bm):
    n, hid = g1.shape
    grid = n // bm

    def body(sp_ref, g1_ref, dinv_ref, g2_ref):
        s = sp_ref[0] + sp_ref[1] + g1_ref[...]
        t = jnp.maximum(s * dinv_ref[...], 0.0)
        g2_ref[...] = t * dinv_ref[...]

    return pl.pallas_call(
        body,
        grid=(grid,),
        in_specs=[
            pl.BlockSpec((NC, bm, hid), lambda i: (0, i, 0)),
            pl.BlockSpec((bm, hid), lambda i: (i, 0)),
            pl.BlockSpec((bm, 1), lambda i: (i, 0)),
        ],
        out_specs=pl.BlockSpec((bm, hid), lambda i: (i, 0)),
        out_shape=jax.ShapeDtypeStruct((n, hid), F32),
    )(s1_p, g1, dinv)


def _stage3(s2_p, g2, dinv, batch2d, W2, n_graphs, bm):
    n, hid = g2.shape
    n_class = W2.shape[1]
    grid = n // bm

    def body(sp_ref, g2_ref, dinv_ref, b_ref, w_ref, out_ref, sums, counts):
        i = pl.program_id(0)

        @pl.when(i == 0)
        def _init():
            sums[...] = jnp.zeros_like(sums)
            counts[...] = jnp.zeros_like(counts)

        val = (sp_ref[0] + sp_ref[1] + g2_ref[...]) * dinv_ref[...]
        gids = lax.broadcasted_iota(jnp.int32, (1, n_graphs), 1)
        onehot = (b_ref[...] == gids).astype(F32)
        sums[...] += lax.dot_general(
            onehot, val, (((0,), (0,)), ((), ())),
            preferred_element_type=F32)
        counts[...] += lax.dot_general(
            onehot, jnp.ones((bm, 1), F32), (((0,), (0,)), ((), ())),
            preferred_element_type=F32)

        @pl.when(i == grid - 1)
        def _fin():
            mean = sums[...] / jnp.maximum(counts[...], 1.0)
            out_ref[...] = jnp.dot(mean, w_ref[...],
                                   preferred_element_type=F32)

    return pl.pallas_call(
        body,
        grid=(grid,),
        in_specs=[
            pl.BlockSpec((NC, bm, hid), lambda i: (0, i, 0)),
            pl.BlockSpec((bm, hid), lambda i: (i, 0)),
            pl.BlockSpec((bm, 1), lambda i: (i, 0)),
            pl.BlockSpec((bm, 1), lambda i: (i, 0)),
            pl.BlockSpec((hid, n_class), lambda i: (0, 0)),
        ],
        out_specs=pl.BlockSpec((n_graphs, n_class), lambda i: (0, 0)),
        out_shape=jax.ShapeDtypeStruct((n_graphs, n_class), F32),
        scratch_shapes=[
            pltpu.VMEM((n_graphs, hid), F32),
            pltpu.VMEM((n_graphs, 1), F32),
        ],
    )(s2_p, g2, dinv, batch2d, W2)


def kernel(x, edge_index, batch, W1, W2):
    n, f = x.shape
    n_edges = edge_index.shape[1]
    hid = W1.shape[1]
    n_graphs = 128
    chunk = 128
    np_ = 10240
    bm = 2048
    ep = 327680
    epw = ep // NW
    n_chunks = epw // chunk

    ei = edge_index.astype(jnp.int32)
    pad_e = n + jnp.arange(ep - n_edges, dtype=jnp.int32) % (np_ - n)
    srcp = jnp.concatenate([ei[0], pad_e])
    dstp = jnp.concatenate([ei[1], pad_e])
    src3 = srcp.reshape(NW, n_chunks, chunk)
    dst3 = dstp.reshape(NW, n_chunks, chunk)
    dst2 = dstp.reshape(NW, epw)
    zh = jnp.zeros((np_, hid), F32)
    xp = jnp.zeros((np_, f), F32).at[:n, :].set(x)
    batch2d = jnp.full((np_, 1), n_graphs, jnp.int32).at[:n, 0].set(
        batch.astype(jnp.int32))

    deg_p = _make_deg_kernel(np_, epw)(dst2)
    h1 = _matmul(xp, W1, bm)
    g1, dinv = _stage1(deg_p, h1, bm)
    agg = _make_edge_agg_kernel(np_, n_chunks, hid)
    s1_p = agg(g1, src3, dst3, zh)
    g2 = _stage2(s1_p, g1, dinv, bm)
    s2_p = agg(g2, src3, dst3, zh)
    return _stage3(s2_p, g2, dinv, batch2d, W2, n_graphs, bm)


# speedup vs baseline: 1.0066x; 1.0066x over previous
import functools

import jax
import jax.numpy as jnp
from jax import lax
from jax.experimental import pallas as pl
from jax.experimental.pallas import tpu as pltpu
from jax.experimental.pallas import tpu_sc as plsc

NC = 2
NS = 16
NW = NC * NS
F32 = jnp.float32


def _make_deg_kernel(np_, epw):
    n16 = epw // 16
    mesh = plsc.VectorSubcoreMesh(core_axis_name="c", subcore_axis_name="s")

    @functools.partial(
        pl.kernel,
        out_type=jax.ShapeDtypeStruct((NW, np_), F32),
        mesh=mesh,
        scratch_types=[
            pltpu.VMEM((epw,), jnp.int32),
            pltpu.VMEM((np_,), F32),
        ],
        compiler_params=pltpu.CompilerParams(needs_layout_passes=False),
    )
    def deg_kernel(dst2, out_hbm, dstv, hist):
        c = lax.axis_index("c")
        s = lax.axis_index("s")
        wid = c * NS + s
        pltpu.sync_copy(dst2.at[wid], dstv)
        zero16 = jnp.zeros((16,), F32)
        ones16 = jnp.ones((16,), F32)

        def zbody(i, carry):
            hist[pl.ds(i * 16, 16)] = zero16
            return carry

        lax.fori_loop(0, np_ // 16, zbody, 0)

        def body(i, carry):
            idx16 = dstv[pl.ds(i * 16, 16)]
            plsc.addupdate_scatter(hist, [idx16], ones16)
            return carry

        lax.fori_loop(0, n16, body, 0)
        pltpu.sync_copy(hist, out_hbm.at[wid])

    return deg_kernel


def _make_edge_agg_kernel(np_, n_chunks, feat, grp=8):
    rpt = np_ // NS
    chunk = 128
    n_grp = n_chunks // grp
    co = 128
    n_co = rpt // co
    mesh = plsc.VectorSubcoreMesh(core_axis_name="c", subcore_axis_name="s")

    @functools.partial(
        pl.kernel,
        out_type=jax.ShapeDtypeStruct((NC, np_, feat), F32),
        mesh=mesh,
        scratch_types=[
            pltpu.VMEM((n_chunks, chunk), jnp.int32),
            pltpu.VMEM((grp, chunk), jnp.int32),
            pltpu.VMEM((2, chunk, feat), F32),
            pltpu.VMEM_SHARED((np_, feat), F32),
            pltpu.SemaphoreType.DMA,
            pltpu.SemaphoreType.DMA,
            pltpu.SemaphoreType.DMA,
            pltpu.SemaphoreType.DMA,
        ],
    )
    def agg_kernel(g_hbm, src3, dst3, zeros_hbm, out_hbm,
                   idx_s, idx_d, rows, acc, sg0, sg1, ss0, ss1):
        c = lax.axis_index("c")
        s = lax.axis_index("s")
        r0 = pl.multiple_of(s * rpt, 128)
        pltpu.sync_copy(zeros_hbm.at[pl.ds(r0, rpt)], acc.at[pl.ds(r0, rpt)])
        wid = c * NS + s
        pltpu.sync_copy(src3.at[wid], idx_s)
        sg = (sg0, sg1)
        ss = (ss0, ss1)

        gd = [None, None]
        gd[0] = pltpu.async_copy(g_hbm.at[idx_s.at[0]], rows.at[0], sg0)
        plsc.subcore_barrier()

        def group(jo, carry):
            j0 = pl.multiple_of(jo * grp, grp)
            pltpu.sync_copy(dst3.at[wid, pl.ds(j0, grp)], idx_d)
            sd = [None] * grp
            for k in range(grp):
                b = k % 2
                if k >= 1:
                    sd[k - 1].wait()
                if k < grp - 1:
                    gd[1 - b] = pltpu.async_copy(
                        g_hbm.at[idx_s.at[jo * grp + k + 1]],
                        rows.at[1 - b], sg[1 - b])
                else:
                    @pl.when(jo < n_grp - 1)
                    def _pref():
                        pltpu.async_copy(
                            g_hbm.at[idx_s.at[jo * grp + grp]],
                            rows.at[1 - b], sg[1 - b])
                gd[b].wait()
                sd[k] = pltpu.async_copy(
                    rows.at[b], acc.at[idx_d.at[k]], ss[b], add=True)
            sd[grp - 1].wait()
            return carry

        lax.fori_loop(0, n_grp, group, 0)
        plsc.subcore_barrier()
        pltpu.sync_copy(acc.at[pl.ds(r0, rpt)], out_hbm.at[c, pl.ds(r0, rpt)])

    return agg_kernel


def _stage1(deg_p, xp, W1, bm):
    n, f = xp.shape
    grid = n // bm
    hid = W1.shape[1]

    def body(degp_ref, x_ref, w_ref, g1_ref, dinv_ref):
        deg = lax.dot_general(
            degp_ref[...], jnp.ones((NW, 1), F32), (((0,), (0,)), ((), ())),
            preferred_element_type=F32) + 1.0
        dinv = lax.rsqrt(deg)
        h = jnp.dot(x_ref[...], w_ref[...], preferred_element_type=F32)
        g1_ref[...] = h * dinv
        dinv_ref[...] = dinv

    return pl.pallas_call(
        body,
        grid=(grid,),
        in_specs=[
            pl.BlockSpec((NW, bm), lambda i: (0, i)),
            pl.BlockSpec((bm, f), lambda i: (i, 0)),
            pl.BlockSpec((f, hid), lambda i: (0, 0)),
        ],
        out_specs=[
            pl.BlockSpec((bm, hid), lambda i: (i, 0)),
            pl.BlockSpec((bm, 1), lambda i: (i, 0)),
        ],
        out_shape=[
            jax.ShapeDtypeStruct((n, hid), F32),
            jax.ShapeDtypeStruct((n, 1), F32),
        ],
    )(deg_p, xp, W1)


def _stage2(s1_p, g1, dinv, bm):
    n, hid = g1.shape
    grid = n // bm

    def body(sp_ref, g1_ref, dinv_ref, g2_ref):
        s = sp_ref[0] + sp_ref[1] + g1_ref[...]
        t = jnp.maximum(s * dinv_ref[...], 0.0)
        g2_ref[...] = t * dinv_ref[...]

    return pl.pallas_call(
        body,
        grid=(grid,),
        in_specs=[
            pl.BlockSpec((NC, bm, hid), lambda i: (0, i, 0)),
            pl.BlockSpec((bm, hid), lambda i: (i, 0)),
            pl.BlockSpec((bm, 1), lambda i: (i, 0)),
        ],
        out_specs=pl.BlockSpec((bm, hid), lambda i: (i, 0)),
        out_shape=jax.ShapeDtypeStruct((n, hid), F32),
    )(s1_p, g1, dinv)


def _stage3(s2_p, g2, dinv, batch2d, W2, n_graphs, bm):
    n, hid = g2.shape
    n_class = W2.shape[1]
    grid = n // bm

    def body(sp_ref, g2_ref, dinv_ref, b_ref, w_ref, out_ref, sums, counts):
        i = pl.program_id(0)

        @pl.when(i == 0)
        def _init():
            sums[...] = jnp.zeros_like(sums)
            counts[...] = jnp.zeros_like(counts)

        val = (sp_ref[0] + sp_ref[1] + g2_ref[...]) * dinv_ref[...]
        gids = lax.broadcasted_iota(jnp.int32, (1, n_graphs), 1)
        onehot = (b_ref[...] == gids).astype(F32)
        sums[...] += lax.dot_general(
            onehot, val, (((0,), (0,)), ((), ())),
            preferred_element_type=F32)
        counts[...] += lax.dot_general(
            onehot, jnp.ones((bm, 1), F32), (((0,), (0,)), ((), ())),
            preferred_element_type=F32)

        @pl.when(i == grid - 1)
        def _fin():
            mean = sums[...] / jnp.maximum(counts[...], 1.0)
            out_ref[...] = jnp.dot(mean, w_ref[...],
                                   preferred_element_type=F32)

    return pl.pallas_call(
        body,
        grid=(grid,),
        in_specs=[
            pl.BlockSpec((NC, bm, hid), lambda i: (0, i, 0)),
            pl.BlockSpec((bm, hid), lambda i: (i, 0)),
            pl.BlockSpec((bm, 1), lambda i: (i, 0)),
            pl.BlockSpec((bm, 1), lambda i: (i, 0)),
            pl.BlockSpec((hid, n_class), lambda i: (0, 0)),
        ],
        out_specs=pl.BlockSpec((n_graphs, n_class), lambda i: (0, 0)),
        out_shape=jax.ShapeDtypeStruct((n_graphs, n_class), F32),
        scratch_shapes=[
            pltpu.VMEM((n_graphs, hid), F32),
            pltpu.VMEM((n_graphs, 1), F32),
        ],
    )(s2_p, g2, dinv, batch2d, W2)


def kernel(x, edge_index, batch, W1, W2):
    n, f = x.shape
    n_edges = edge_index.shape[1]
    hid = W1.shape[1]
    n_graphs = 128
    chunk = 128
    np_ = 10240
    bm = 2048
    ep = 327680
    epw = ep // NW
    n_chunks = epw // chunk

    ei = edge_index.astype(jnp.int32)
    pad_e = n + jnp.arange(ep - n_edges, dtype=jnp.int32) % (np_ - n)
    srcp = jnp.concatenate([ei[0], pad_e])
    dstp = jnp.concatenate([ei[1], pad_e])
    src3 = srcp.reshape(NW, n_chunks, chunk)
    dst3 = dstp.reshape(NW, n_chunks, chunk)
    dst2 = dstp.reshape(NW, epw)
    zh = jnp.zeros((np_, hid), F32)
    xp = jnp.zeros((np_, f), F32).at[:n, :].set(x)
    batch2d = jnp.full((np_, 1), n_graphs, jnp.int32).at[:n, 0].set(
        batch.astype(jnp.int32))

    deg_p = _make_deg_kernel(np_, epw)(dst2)
    g1, dinv = _stage1(deg_p, xp, W1, bm)
    agg = _make_edge_agg_kernel(np_, n_chunks, hid)
    s1_p = agg(g1, src3, dst3, zh)
    g2 = _stage2(s1_p, g1, dinv, bm)
    s2_p = agg(g2, src3, dst3, zh)
    return _stage3(s2_p, g2, dinv, batch2d, W2, n_graphs, bm)


# speedup vs baseline: 1.0227x; 1.0159x over previous
import functools

import jax
import jax.numpy as jnp
from jax import lax
from jax.experimental import pallas as pl
from jax.experimental.pallas import tpu as pltpu
from jax.experimental.pallas import tpu_sc as plsc

NC = 2
NS = 16
NW = NC * NS
F32 = jnp.float32


def _make_deg_kernel(np_, epw):
    n16 = epw // 16
    mesh = plsc.VectorSubcoreMesh(core_axis_name="c", subcore_axis_name="s")

    @functools.partial(
        pl.kernel,
        out_type=jax.ShapeDtypeStruct((NW, np_), F32),
        mesh=mesh,
        scratch_types=[
            pltpu.VMEM((epw,), jnp.int32),
            pltpu.VMEM((np_,), F32),
        ],
        compiler_params=pltpu.CompilerParams(needs_layout_passes=False),
    )
    def deg_kernel(dst2, out_hbm, dstv, hist):
        c = lax.axis_index("c")
        s = lax.axis_index("s")
        wid = c * NS + s
        pltpu.sync_copy(dst2.at[wid], dstv)
        zero16 = jnp.zeros((16,), F32)
        ones16 = jnp.ones((16,), F32)

        def zbody(i, carry):
            hist[pl.ds(i * 16, 16)] = zero16
            return carry

        lax.fori_loop(0, np_ // 16, zbody, 0)

        def body(i, carry):
            idx16 = dstv[pl.ds(i * 16, 16)]
            plsc.addupdate_scatter(hist, [idx16], ones16)
            return carry

        lax.fori_loop(0, n16, body, 0)
        pltpu.sync_copy(hist, out_hbm.at[wid])

    return deg_kernel


def _make_edge_agg_kernel(np_, n_chunks, feat, grp=16):
    rpt = np_ // NS
    chunk = 128
    n_grp = n_chunks // grp
    co = 128
    n_co = rpt // co
    mesh = plsc.VectorSubcoreMesh(core_axis_name="c", subcore_axis_name="s")

    @functools.partial(
        pl.kernel,
        out_type=jax.ShapeDtypeStruct((NC, np_, feat), F32),
        mesh=mesh,
        scratch_types=[
            pltpu.VMEM((n_chunks, chunk), jnp.int32),
            pltpu.VMEM((grp, chunk), jnp.int32),
            pltpu.VMEM((2, chunk, feat), F32),
            pltpu.VMEM_SHARED((np_, feat), F32),
            pltpu.SemaphoreType.DMA,
            pltpu.SemaphoreType.DMA,
            pltpu.SemaphoreType.DMA,
            pltpu.SemaphoreType.DMA,
        ],
    )
    def agg_kernel(g_hbm, src3, dst3, zeros_hbm, out_hbm,
                   idx_s, idx_d, rows, acc, sg0, sg1, ss0, ss1):
        c = lax.axis_index("c")
        s = lax.axis_index("s")
        r0 = pl.multiple_of(s * rpt, 128)
        pltpu.sync_copy(zeros_hbm.at[pl.ds(r0, rpt)], acc.at[pl.ds(r0, rpt)])
        wid = c * NS + s
        pltpu.sync_copy(src3.at[wid], idx_s)
        sg = (sg0, sg1)
        ss = (ss0, ss1)

        gd = [None, None]
        gd[0] = pltpu.async_copy(g_hbm.at[idx_s.at[0]], rows.at[0], sg0)
        plsc.subcore_barrier()

        def group(jo, carry):
            j0 = pl.multiple_of(jo * grp, grp)
            pltpu.sync_copy(dst3.at[wid, pl.ds(j0, grp)], idx_d)
            sd = [None] * grp
            for k in range(grp):
                b = k % 2
                if k >= 1:
                    sd[k - 1].wait()
                if k < grp - 1:
                    gd[1 - b] = pltpu.async_copy(
                        g_hbm.at[idx_s.at[jo * grp + k + 1]],
                        rows.at[1 - b], sg[1 - b])
                else:
                    @pl.when(jo < n_grp - 1)
                    def _pref():
                        pltpu.async_copy(
                            g_hbm.at[idx_s.at[jo * grp + grp]],
                            rows.at[1 - b], sg[1 - b])
                gd[b].wait()
                sd[k] = pltpu.async_copy(
                    rows.at[b], acc.at[idx_d.at[k]], ss[b], add=True)
            sd[grp - 1].wait()
            return carry

        lax.fori_loop(0, n_grp, group, 0)
        plsc.subcore_barrier()
        pltpu.sync_copy(acc.at[pl.ds(r0, rpt)], out_hbm.at[c, pl.ds(r0, rpt)])

    return agg_kernel


def _stage1(deg_p, xp, W1, bm):
    n, f = xp.shape
    grid = n // bm
    hid = W1.shape[1]

    def body(degp_ref, x_ref, w_ref, g1_ref, dinv_ref):
        deg = lax.dot_general(
            degp_ref[...], jnp.ones((NW, 1), F32), (((0,), (0,)), ((), ())),
            preferred_element_type=F32) + 1.0
        dinv = lax.rsqrt(deg)
        h = jnp.dot(x_ref[...], w_ref[...], preferred_element_type=F32)
        g1_ref[...] = h * dinv
        dinv_ref[...] = dinv

    return pl.pallas_call(
        body,
        grid=(grid,),
        in_specs=[
            pl.BlockSpec((NW, bm), lambda i: (0, i)),
            pl.BlockSpec((bm, f), lambda i: (i, 0)),
            pl.BlockSpec((f, hid), lambda i: (0, 0)),
        ],
        out_specs=[
            pl.BlockSpec((bm, hid), lambda i: (i, 0)),
            pl.BlockSpec((bm, 1), lambda i: (i, 0)),
        ],
        out_shape=[
            jax.ShapeDtypeStruct((n, hid), F32),
            jax.ShapeDtypeStruct((n, 1), F32),
        ],
    )(deg_p, xp, W1)


def _stage2(s1_p, g1, dinv, bm):
    n, hid = g1.shape
    grid = n // bm

    def body(sp_ref, g1_ref, dinv_ref, g2_ref):
        s = sp_ref[0] + sp_ref[1] + g1_ref[...]
        t = jnp.maximum(s * dinv_ref[...], 0.0)
        g2_ref[...] = t * dinv_ref[...]

    return pl.pallas_call(
        body,
        grid=(grid,),
        in_specs=[
            pl.BlockSpec((NC, bm, hid), lambda i: (0, i, 0)),
            pl.BlockSpec((bm, hid), lambda i: (i, 0)),
            pl.BlockSpec((bm, 1), lambda i: (i, 0)),
        ],
        out_specs=pl.BlockSpec((bm, hid), lambda i: (i, 0)),
        out_shape=jax.ShapeDtypeStruct((n, hid), F32),
    )(s1_p, g1, dinv)


def _stage3(s2_p, g2, dinv, batch2d, W2, n_graphs, bm):
    n, hid = g2.shape
    n_class = W2.shape[1]
    grid = n // bm

    def body(sp_ref, g2_ref, dinv_ref, b_ref, w_ref, out_ref, sums, counts):
        i = pl.program_id(0)

        @pl.when(i == 0)
        def _init():
            sums[...] = jnp.zeros_like(sums)
            counts[...] = jnp.zeros_like(counts)

        val = (sp_ref[0] + sp_ref[1] + g2_ref[...]) * dinv_ref[...]
        gids = lax.broadcasted_iota(jnp.int32, (1, n_graphs), 1)
        onehot = (b_ref[...] == gids).astype(F32)
        sums[...] += lax.dot_general(
            onehot, val, (((0,), (0,)), ((), ())),
            preferred_element_type=F32)
        counts[...] += lax.dot_general(
            onehot, jnp.ones((bm, 1), F32), (((0,), (0,)), ((), ())),
            preferred_element_type=F32)

        @pl.when(i == grid - 1)
        def _fin():
            mean = sums[...] / jnp.maximum(counts[...], 1.0)
            out_ref[...] = jnp.dot(mean, w_ref[...],
                                   preferred_element_type=F32)

    return pl.pallas_call(
        body,
        grid=(grid,),
        in_specs=[
            pl.BlockSpec((NC, bm, hid), lambda i: (0, i, 0)),
            pl.BlockSpec((bm, hid), lambda i: (i, 0)),
            pl.BlockSpec((bm, 1), lambda i: (i, 0)),
            pl.BlockSpec((bm, 1), lambda i: (i, 0)),
            pl.BlockSpec((hid, n_class), lambda i: (0, 0)),
        ],
        out_specs=pl.BlockSpec((n_graphs, n_class), lambda i: (0, 0)),
        out_shape=jax.ShapeDtypeStruct((n_graphs, n_class), F32),
        scratch_shapes=[
            pltpu.VMEM((n_graphs, hid), F32),
            pltpu.VMEM((n_graphs, 1), F32),
        ],
    )(s2_p, g2, dinv, batch2d, W2)


def kernel(x, edge_index, batch, W1, W2):
    n, f = x.shape
    n_edges = edge_index.shape[1]
    hid = W1.shape[1]
    n_graphs = 128
    chunk = 128
    np_ = 10240
    bm = 2048
    ep = 327680
    epw = ep // NW
    n_chunks = epw // chunk

    ei = edge_index.astype(jnp.int32)
    pad_e = n + jnp.arange(ep - n_edges, dtype=jnp.int32) % (np_ - n)
    srcp = jnp.concatenate([ei[0], pad_e])
    dstp = jnp.concatenate([ei[1], pad_e])
    src3 = srcp.reshape(NW, n_chunks, chunk)
    dst3 = dstp.reshape(NW, n_chunks, chunk)
    dst2 = dstp.reshape(NW, epw)
    zh = jnp.zeros((np_, hid), F32)
    xp = jnp.zeros((np_, f), F32).at[:n, :].set(x)
    batch2d = jnp.full((np_, 1), n_graphs, jnp.int32).at[:n, 0].set(
        batch.astype(jnp.int32))

    deg_p = _make_deg_kernel(np_, epw)(dst2)
    g1, dinv = _stage1(deg_p, xp, W1, bm)
    agg = _make_edge_agg_kernel(np_, n_chunks, hid)
    s1_p = agg(g1, src3, dst3, zh)
    g2 = _stage2(s1_p, g1, dinv, bm)
    s2_p = agg(g2, src3, dst3, zh)
    return _stage3(s2_p, g2, dinv, batch2d, W2, n_graphs, bm)


# speedup vs baseline: 1.0323x; 1.0094x over previous
import functools

import jax
import jax.numpy as jnp
from jax import lax
from jax.experimental import pallas as pl
from jax.experimental.pallas import tpu as pltpu
from jax.experimental.pallas import tpu_sc as plsc

NC = 2
NS = 16
NW = NC * NS
F32 = jnp.float32


def _make_deg_kernel(np_, epw):
    n16 = epw // 16
    mesh = plsc.VectorSubcoreMesh(core_axis_name="c", subcore_axis_name="s")

    @functools.partial(
        pl.kernel,
        out_type=jax.ShapeDtypeStruct((NW, np_), F32),
        mesh=mesh,
        scratch_types=[
            pltpu.VMEM((epw,), jnp.int32),
            pltpu.VMEM((np_,), F32),
        ],
        compiler_params=pltpu.CompilerParams(needs_layout_passes=False),
    )
    def deg_kernel(dst2, out_hbm, dstv, hist):
        c = lax.axis_index("c")
        s = lax.axis_index("s")
        wid = c * NS + s
        pltpu.sync_copy(dst2.at[wid], dstv)
        zero16 = jnp.zeros((16,), F32)
        ones16 = jnp.ones((16,), F32)

        def zbody(i, carry):
            hist[pl.ds(i * 16, 16)] = zero16
            return carry

        lax.fori_loop(0, np_ // 16, zbody, 0)

        def body(i, carry):
            idx16 = dstv[pl.ds(i * 16, 16)]
            plsc.addupdate_scatter(hist, [idx16], ones16)
            return carry

        lax.fori_loop(0, n16, body, 0)
        pltpu.sync_copy(hist, out_hbm.at[wid])

    return deg_kernel


def _make_edge_agg_kernel(np_, n_chunks, feat, grp=16):
    rpt = np_ // NS
    chunk = 128
    n_grp = n_chunks // grp
    co = 128
    n_co = rpt // co
    mesh = plsc.VectorSubcoreMesh(core_axis_name="c", subcore_axis_name="s")

    @functools.partial(
        pl.kernel,
        out_type=jax.ShapeDtypeStruct((NC, np_, feat), F32),
        mesh=mesh,
        scratch_types=[
            pltpu.VMEM((n_chunks, chunk), jnp.int32),
            pltpu.VMEM((grp, chunk), jnp.int32),
            pltpu.VMEM((2, chunk, feat), F32),
            pltpu.VMEM_SHARED((np_, feat), F32),
            pltpu.SemaphoreType.DMA,
            pltpu.SemaphoreType.DMA,
            pltpu.SemaphoreType.DMA,
            pltpu.SemaphoreType.DMA,
            pltpu.SemaphoreType.DMA,
        ],
    )
    def agg_kernel(g_hbm, src3, dst3, zeros_hbm, out_hbm,
                   idx_s, idx_d, rows, acc, sg0, sg1, ss0, ss1, si):
        c = lax.axis_index("c")
        s = lax.axis_index("s")
        r0 = pl.multiple_of(s * rpt, 128)
        pltpu.sync_copy(zeros_hbm.at[pl.ds(r0, rpt)], acc.at[pl.ds(r0, rpt)])
        wid = c * NS + s
        pltpu.sync_copy(src3.at[wid], idx_s)
        sg = (sg0, sg1)
        ss = (ss0, ss1)

        gd = [None, None]
        gd[0] = pltpu.async_copy(g_hbm.at[idx_s.at[0]], rows.at[0], sg0)
        plsc.subcore_barrier()

        def group(jo, carry):
            j0 = pl.multiple_of(jo * grp, grp)
            dd = pltpu.async_copy(dst3.at[wid, pl.ds(j0, grp)], idx_d, si)
            sd = [None] * grp
            for k in range(grp):
                b = k % 2
                if k >= 1:
                    sd[k - 1].wait()
                if k < grp - 1:
                    gd[1 - b] = pltpu.async_copy(
                        g_hbm.at[idx_s.at[jo * grp + k + 1]],
                        rows.at[1 - b], sg[1 - b])
                else:
                    @pl.when(jo < n_grp - 1)
                    def _pref():
                        pltpu.async_copy(
                            g_hbm.at[idx_s.at[jo * grp + grp]],
                            rows.at[1 - b], sg[1 - b])
                gd[b].wait()
                if k == 0:
                    dd.wait()
                sd[k] = pltpu.async_copy(
                    rows.at[b], acc.at[idx_d.at[k]], ss[b], add=True)
            sd[grp - 1].wait()
            return carry

        lax.fori_loop(0, n_grp, group, 0)
        plsc.subcore_barrier()
        pltpu.sync_copy(acc.at[pl.ds(r0, rpt)], out_hbm.at[c, pl.ds(r0, rpt)])

    return agg_kernel


def _stage1(deg_p, xp, W1, bm):
    n, f = xp.shape
    grid = n // bm
    hid = W1.shape[1]

    def body(degp_ref, x_ref, w_ref, g1_ref, dinv_ref):
        deg = lax.dot_general(
            degp_ref[...], jnp.ones((NW, 1), F32), (((0,), (0,)), ((), ())),
            preferred_element_type=F32) + 1.0
        dinv = lax.rsqrt(deg)
        h = jnp.dot(x_ref[...], w_ref[...], preferred_element_type=F32)
        g1_ref[...] = h * dinv
        dinv_ref[...] = dinv

    return pl.pallas_call(
        body,
        grid=(grid,),
        in_specs=[
            pl.BlockSpec((NW, bm), lambda i: (0, i)),
            pl.BlockSpec((bm, f), lambda i: (i, 0)),
            pl.BlockSpec((f, hid), lambda i: (0, 0)),
        ],
        out_specs=[
            pl.BlockSpec((bm, hid), lambda i: (i, 0)),
            pl.BlockSpec((bm, 1), lambda i: (i, 0)),
        ],
        out_shape=[
            jax.ShapeDtypeStruct((n, hid), F32),
            jax.ShapeDtypeStruct((n, 1), F32),
        ],
    )(deg_p, xp, W1)


def _stage2(s1_p, g1, dinv, bm):
    n, hid = g1.shape
    grid = n // bm

    def body(sp_ref, g1_ref, dinv_ref, g2_ref):
        s = sp_ref[0] + sp_ref[1] + g1_ref[...]
        t = jnp.maximum(s * dinv_ref[...], 0.0)
        g2_ref[...] = t * dinv_ref[...]

    return pl.pallas_call(
        body,
        grid=(grid,),
        in_specs=[
            pl.BlockSpec((NC, bm, hid), lambda i: (0, i, 0)),
            pl.BlockSpec((bm, hid), lambda i: (i, 0)),
            pl.BlockSpec((bm, 1), lambda i: (i, 0)),
        ],
        out_specs=pl.BlockSpec((bm, hid), lambda i: (i, 0)),
        out_shape=jax.ShapeDtypeStruct((n, hid), F32),
    )(s1_p, g1, dinv)


def _stage3(s2_p, g2, dinv, batch2d, W2, n_graphs, bm):
    n, hid = g2.shape
    n_class = W2.shape[1]
    grid = n // bm

    def body(sp_ref, g2_ref, dinv_ref, b_ref, w_ref, out_ref, sums, counts):
        i = pl.program_id(0)

        @pl.when(i == 0)
        def _init():
            sums[...] = jnp.zeros_like(sums)
            counts[...] = jnp.zeros_like(counts)

        val = (sp_ref[0] + sp_ref[1] + g2_ref[...]) * dinv_ref[...]
        gids = lax.broadcasted_iota(jnp.int32, (1, n_graphs), 1)
        onehot = (b_ref[...] == gids).astype(F32)
        sums[...] += lax.dot_general(
            onehot, val, (((0,), (0,)), ((), ())),
            preferred_element_type=F32)
        counts[...] += lax.dot_general(
            onehot, jnp.ones((bm, 1), F32), (((0,), (0,)), ((), ())),
            preferred_element_type=F32)

        @pl.when(i == grid - 1)
        def _fin():
            mean = sums[...] / jnp.maximum(counts[...], 1.0)
            out_ref[...] = jnp.dot(mean, w_ref[...],
                                   preferred_element_type=F32)

    return pl.pallas_call(
        body,
        grid=(grid,),
        in_specs=[
            pl.BlockSpec((NC, bm, hid), lambda i: (0, i, 0)),
            pl.BlockSpec((bm, hid), lambda i: (i, 0)),
            pl.BlockSpec((bm, 1), lambda i: (i, 0)),
            pl.BlockSpec((bm, 1), lambda i: (i, 0)),
            pl.BlockSpec((hid, n_class), lambda i: (0, 0)),
        ],
        out_specs=pl.BlockSpec((n_graphs, n_class), lambda i: (0, 0)),
        out_shape=jax.ShapeDtypeStruct((n_graphs, n_class), F32),
        scratch_shapes=[
            pltpu.VMEM((n_graphs, hid), F32),
            pltpu.VMEM((n_graphs, 1), F32),
        ],
    )(s2_p, g2, dinv, batch2d, W2)


def kernel(x, edge_index, batch, W1, W2):
    n, f = x.shape
    n_edges = edge_index.shape[1]
    hid = W1.shape[1]
    n_graphs = 128
    chunk = 128
    np_ = 10240
    bm = 2048
    ep = 327680
    epw = ep // NW
    n_chunks = epw // chunk

    ei = edge_index.astype(jnp.int32)
    pad_e = n + jnp.arange(ep - n_edges, dtype=jnp.int32) % (np_ - n)
    srcp = jnp.concatenate([ei[0], pad_e])
    dstp = jnp.concatenate([ei[1], pad_e])
    src3 = srcp.reshape(NW, n_chunks, chunk)
    dst3 = dstp.reshape(NW, n_chunks, chunk)
    dst2 = dstp.reshape(NW, epw)
    zh = jnp.zeros((np_, hid), F32)
    xp = jnp.zeros((np_, f), F32).at[:n, :].set(x)
    batch2d = jnp.full((np_, 1), n_graphs, jnp.int32).at[:n, 0].set(
        batch.astype(jnp.int32))

    deg_p = _make_deg_kernel(np_, epw)(dst2)
    g1, dinv = _stage1(deg_p, xp, W1, bm)
    agg = _make_edge_agg_kernel(np_, n_chunks, hid)
    s1_p = agg(g1, src3, dst3, zh)
    g2 = _stage2(s1_p, g1, dinv, bm)
    s2_p = agg(g2, src3, dst3, zh)
    return _stage3(s2_p, g2, dinv, batch2d, W2, n_graphs, bm)


# speedup vs baseline: 1.0444x; 1.0118x over previous
import functools

import jax
import jax.numpy as jnp
from jax import lax
from jax.experimental import pallas as pl
from jax.experimental.pallas import tpu as pltpu
from jax.experimental.pallas import tpu_sc as plsc

NC = 2
NS = 16
NW = NC * NS
F32 = jnp.float32


def _make_deg_kernel(np_, epw):
    n16 = epw // 16
    mesh = plsc.VectorSubcoreMesh(core_axis_name="c", subcore_axis_name="s")

    @functools.partial(
        pl.kernel,
        out_type=jax.ShapeDtypeStruct((NW, np_), F32),
        mesh=mesh,
        scratch_types=[
            pltpu.VMEM((epw,), jnp.int32),
            pltpu.VMEM((np_,), F32),
        ],
        compiler_params=pltpu.CompilerParams(needs_layout_passes=False),
    )
    def deg_kernel(dst2, out_hbm, dstv, hist):
        c = lax.axis_index("c")
        s = lax.axis_index("s")
        wid = c * NS + s
        pltpu.sync_copy(dst2.at[wid], dstv)
        zero16 = jnp.zeros((16,), F32)
        ones16 = jnp.ones((16,), F32)

        def zbody(i, carry):
            hist[pl.ds(i * 16, 16)] = zero16
            return carry

        lax.fori_loop(0, np_ // 16, zbody, 0)

        def body(i, carry):
            idx16 = dstv[pl.ds(i * 16, 16)]
            plsc.addupdate_scatter(hist, [idx16], ones16)
            return carry

        lax.fori_loop(0, n16, body, 0)
        pltpu.sync_copy(hist, out_hbm.at[wid])

    return deg_kernel


def _make_edge_agg_kernel(np_, n_chunks, feat, grp=16):
    rpt = np_ // NS
    chunk = 128
    n_grp = n_chunks // grp
    co = 128
    n_co = rpt // co
    mesh = plsc.VectorSubcoreMesh(core_axis_name="c", subcore_axis_name="s")

    @functools.partial(
        pl.kernel,
        out_type=jax.ShapeDtypeStruct((NC, np_, feat), F32),
        mesh=mesh,
        scratch_types=[
            pltpu.VMEM((n_chunks, chunk), jnp.int32),
            pltpu.VMEM((grp, chunk), jnp.int32),
            pltpu.VMEM((2, chunk, feat), F32),
            pltpu.VMEM_SHARED((np_, feat), F32),
            pltpu.SemaphoreType.DMA,
            pltpu.SemaphoreType.DMA,
            pltpu.SemaphoreType.DMA,
            pltpu.SemaphoreType.DMA,
            pltpu.SemaphoreType.DMA,
        ],
    )
    def agg_kernel(g_hbm, src3, dst3, zeros_hbm, out_hbm,
                   idx_s, idx_d, rows, acc, sg0, sg1, ss0, ss1, si):
        c = lax.axis_index("c")
        s = lax.axis_index("s")
        r0 = pl.multiple_of(s * rpt, 128)
        pltpu.sync_copy(zeros_hbm.at[pl.ds(r0, rpt)], acc.at[pl.ds(r0, rpt)])
        wid = c * NS + s
        pltpu.sync_copy(src3.at[wid], idx_s)
        sg = (sg0, sg1)
        ss = (ss0, ss1)

        gd = [None, None]
        gd[0] = pltpu.async_copy(g_hbm.at[idx_s.at[0]], rows.at[0], sg0)
        plsc.subcore_barrier()

        def group(jo, carry):
            j0 = pl.multiple_of(jo * grp, grp)
            dd = pltpu.async_copy(dst3.at[wid, pl.ds(j0, grp)], idx_d, si)
            sd = [None] * grp
            for k in range(grp):
                b = k % 2
                if k >= 1:
                    sd[k - 1].wait()
                if k < grp - 1:
                    gd[1 - b] = pltpu.async_copy(
                        g_hbm.at[idx_s.at[jo * grp + k + 1]],
                        rows.at[1 - b], sg[1 - b])
                else:
                    @pl.when(jo < n_grp - 1)
                    def _pref():
                        pltpu.async_copy(
                            g_hbm.at[idx_s.at[jo * grp + grp]],
                            rows.at[1 - b], sg[1 - b])
                gd[b].wait()
                if k == 0:
                    dd.wait()
                sd[k] = pltpu.async_copy(
                    rows.at[b], acc.at[idx_d.at[k]], ss[b], add=True)
            sd[grp - 1].wait()
            return carry

        lax.fori_loop(0, n_grp, group, 0)
        plsc.subcore_barrier()
        pltpu.sync_copy(acc.at[pl.ds(r0, rpt)], out_hbm.at[c, pl.ds(r0, rpt)])

    return agg_kernel


def _stage1(deg_p, xp, W1, bm):
    n, f = xp.shape
    grid = n // bm
    hid = W1.shape[1]

    def body(degp_ref, x_ref, w_ref, g1_ref, dinv_ref):
        deg = lax.dot_general(
            degp_ref[...], jnp.ones((NW, 1), F32), (((0,), (0,)), ((), ())),
            preferred_element_type=F32) + 1.0
        dinv = lax.rsqrt(deg)
        h = jnp.dot(x_ref[...], w_ref[...], preferred_element_type=F32)
        g1_ref[...] = h * dinv
        dinv_ref[...] = dinv

    return pl.pallas_call(
        body,
        grid=(grid,),
        in_specs=[
            pl.BlockSpec((NW, bm), lambda i: (0, i)),
            pl.BlockSpec((bm, f), lambda i: (i, 0)),
            pl.BlockSpec((f, hid), lambda i: (0, 0)),
        ],
        out_specs=[
            pl.BlockSpec((bm, hid), lambda i: (i, 0)),
            pl.BlockSpec((bm, 1), lambda i: (i, 0)),
        ],
        out_shape=[
            jax.ShapeDtypeStruct((n, hid), F32),
            jax.ShapeDtypeStruct((n, 1), F32),
        ],
    )(deg_p, xp, W1)


def _stage2(s1_p, g1, dinv, bm):
    n, hid = g1.shape
    grid = n // bm

    def body(sp_ref, g1_ref, dinv_ref, g2_ref):
        s = sp_ref[0] + sp_ref[1] + g1_ref[...]
        t = jnp.maximum(s * dinv_ref[...], 0.0)
        g2_ref[...] = t * dinv_ref[...]

    return pl.pallas_call(
        body,
        grid=(grid,),
        in_specs=[
            pl.BlockSpec((NC, bm, hid), lambda i: (0, i, 0)),
            pl.BlockSpec((bm, hid), lambda i: (i, 0)),
            pl.BlockSpec((bm, 1), lambda i: (i, 0)),
        ],
        out_specs=pl.BlockSpec((bm, hid), lambda i: (i, 0)),
        out_shape=jax.ShapeDtypeStruct((n, hid), F32),
    )(s1_p, g1, dinv)


def _stage3(s2_p, g2, dinv, batch2d, W2, n_graphs, bm):
    n, hid = g2.shape
    n_class = W2.shape[1]
    grid = n // bm

    def body(sp_ref, g2_ref, dinv_ref, b_ref, w_ref, out_ref, sums, counts):
        i = pl.program_id(0)

        @pl.when(i == 0)
        def _init():
            sums[...] = jnp.zeros_like(sums)
            counts[...] = jnp.zeros_like(counts)

        val = (sp_ref[0] + sp_ref[1] + g2_ref[...]) * dinv_ref[...]
        gids = lax.broadcasted_iota(jnp.int32, (1, n_graphs), 1)
        onehot = (b_ref[...] == gids).astype(F32)
        sums[...] += lax.dot_general(
            onehot, val, (((0,), (0,)), ((), ())),
            preferred_element_type=F32)
        counts[...] += lax.dot_general(
            onehot, jnp.ones((bm, 1), F32), (((0,), (0,)), ((), ())),
            preferred_element_type=F32)

        @pl.when(i == grid - 1)
        def _fin():
            mean = sums[...] / jnp.maximum(counts[...], 1.0)
            out_ref[...] = jnp.dot(mean, w_ref[...],
                                   preferred_element_type=F32)

    return pl.pallas_call(
        body,
        grid=(grid,),
        in_specs=[
            pl.BlockSpec((NC, bm, hid), lambda i: (0, i, 0)),
            pl.BlockSpec((bm, hid), lambda i: (i, 0)),
            pl.BlockSpec((bm, 1), lambda i: (i, 0)),
            pl.BlockSpec((bm, 1), lambda i: (i, 0)),
            pl.BlockSpec((hid, n_class), lambda i: (0, 0)),
        ],
        out_specs=pl.BlockSpec((n_graphs, n_class), lambda i: (0, 0)),
        out_shape=jax.ShapeDtypeStruct((n_graphs, n_class), F32),
        scratch_shapes=[
            pltpu.VMEM((n_graphs, hid), F32),
            pltpu.VMEM((n_graphs, 1), F32),
        ],
    )(s2_p, g2, dinv, batch2d, W2)


def kernel(x, edge_index, batch, W1, W2):
    n, f = x.shape
    n_edges = edge_index.shape[1]
    hid = W1.shape[1]
    n_graphs = 128
    chunk = 128
    np_ = 10240
    bm = 5120
    ep = 327680
    epw = ep // NW
    n_chunks = epw // chunk

    ei = edge_index.astype(jnp.int32)
    pad_e = n + jnp.arange(ep - n_edges, dtype=jnp.int32) % (np_ - n)
    srcp = jnp.concatenate([ei[0], pad_e])
    dstp = jnp.concatenate([ei[1], pad_e])
    src3 = srcp.reshape(NW, n_chunks, chunk)
    dst3 = dstp.reshape(NW, n_chunks, chunk)
    dst2 = dstp.reshape(NW, epw)
    zh = jnp.zeros((np_, hid), F32)
    xp = jnp.zeros((np_, f), F32).at[:n, :].set(x)
    batch2d = jnp.full((np_, 1), n_graphs, jnp.int32).at[:n, 0].set(
        batch.astype(jnp.int32))

    deg_p = _make_deg_kernel(np_, epw)(dst2)
    g1, dinv = _stage1(deg_p, xp, W1, bm)
    agg = _make_edge_agg_kernel(np_, n_chunks, hid)
    s1_p = agg(g1, src3, dst3, zh)
    g2 = _stage2(s1_p, g1, dinv, bm)
    s2_p = agg(g2, src3, dst3, zh)
    return _stage3(s2_p, g2, dinv, batch2d, W2, n_graphs, bm)
